# baseline (speedup 1.0000x reference)
import jax
import jax.numpy as jnp
from jax.experimental import pallas as pl
from jax.experimental.pallas import tpu as pltpu

E = 64
H = 768
I = 3072


def _ffn_block_kernel(x_ref, b1_ref, b2_ref, w1_ref, w2_ref, o_ref):
    e = pl.program_id(0)
    per = x_ref.shape[0] // pl.num_programs(0)
    x = x_ref[pl.ds(e * per, per), :]
    h = jnp.dot(x, w1_ref[0], preferred_element_type=jnp.float32)
    h = h + b1_ref[pl.ds(e, 1), :]
    o = jnp.dot(h, w2_ref[0], preferred_element_type=jnp.float32)
    o_ref[pl.ds(e * per, per), :] = o + b2_ref[pl.ds(e, 1), :]


def kernel(hidden_states, W1, b1, W2, b2):
    Bb, Ss, Hh = hidden_states.shape
    Ee = W1.shape[0]
    T = Bb * Ss
    x = hidden_states.reshape(T, Hh)

    out = pl.pallas_call(
        _ffn_block_kernel,
        grid=(Ee,),
        in_specs=[
            pl.BlockSpec((T, Hh), lambda e: (0, 0)),
            pl.BlockSpec((Ee, I), lambda e: (0, 0)),
            pl.BlockSpec((Ee, Hh), lambda e: (0, 0)),
            pl.BlockSpec((1, Hh, I), lambda e: (e, 0, 0)),
            pl.BlockSpec((1, I, Hh), lambda e: (e, 0, 0)),
        ],
        out_specs=pl.BlockSpec((T, Hh), lambda e: (0, 0)),
        out_shape=jax.ShapeDtypeStruct((T, Hh), jnp.float32),
        compiler_params=pltpu.CompilerParams(
            dimension_semantics=("arbitrary",),
        ),
    )(x, b1, b2, W1, W2)
    return out.reshape(Bb, Ss, Hh)


# speedup vs baseline: 1.0209x; 1.0209x over previous
import jax
import jax.numpy as jnp
from jax.experimental import pallas as pl
from jax.experimental.pallas import tpu as pltpu

E = 64
H = 768
I = 3072


HALF = I // 2


def _ffn_block_kernel(x_ref, b1_ref, b2_ref, w1a_ref, w2a_ref, w1b_ref,
                      w2b_ref, o_ref):
    e = pl.program_id(0)
    per = x_ref.shape[0] // pl.num_programs(0)
    x = x_ref[pl.ds(e * per, per), :]
    h1 = jnp.dot(x, w1a_ref[0], preferred_element_type=jnp.float32)
    h1 = h1 + b1_ref[pl.ds(e, 1), :HALF]
    o = jnp.dot(h1, w2a_ref[0], preferred_element_type=jnp.float32)
    h2 = jnp.dot(x, w1b_ref[0], preferred_element_type=jnp.float32)
    h2 = h2 + b1_ref[pl.ds(e, 1), HALF:]
    o = o + jnp.dot(h2, w2b_ref[0], preferred_element_type=jnp.float32)
    o_ref[pl.ds(e * per, per), :] = o + b2_ref[pl.ds(e, 1), :]


def kernel(hidden_states, W1, b1, W2, b2):
    Bb, Ss, Hh = hidden_states.shape
    Ee = W1.shape[0]
    T = Bb * Ss
    x = hidden_states.reshape(T, Hh)

    out = pl.pallas_call(
        _ffn_block_kernel,
        grid=(Ee,),
        in_specs=[
            pl.BlockSpec((T, Hh), lambda e: (0, 0)),
            pl.BlockSpec((Ee, I), lambda e: (0, 0)),
            pl.BlockSpec((Ee, Hh), lambda e: (0, 0)),
            pl.BlockSpec((1, Hh, HALF), lambda e: (e, 0, 0)),
            pl.BlockSpec((1, HALF, Hh), lambda e: (e, 0, 0)),
            pl.BlockSpec((1, Hh, HALF), lambda e: (e, 0, 1)),
            pl.BlockSpec((1, HALF, Hh), lambda e: (e, 1, 0)),
        ],
        out_specs=pl.BlockSpec((T, Hh), lambda e: (0, 0)),
        out_shape=jax.ShapeDtypeStruct((T, Hh), jnp.float32),
        compiler_params=pltpu.CompilerParams(
            dimension_semantics=("arbitrary",),
        ),
    )(x, b1, b2, W1, W2, W1, W2)
    return out.reshape(Bb, Ss, Hh)


# speedup vs baseline: 1.0242x; 1.0033x over previous
import jax
import jax.numpy as jnp
from jax.experimental import pallas as pl
from jax.experimental.pallas import tpu as pltpu

E = 64
H = 768
I = 3072


NSPLIT = 4
CHUNK = I // NSPLIT


def _ffn_block_kernel(x_ref, b1_ref, b2_ref, *w_and_o):
    w_refs = w_and_o[:-1]
    o_ref = w_and_o[-1]
    e = pl.program_id(0)
    per = x_ref.shape[0] // pl.num_programs(0)
    x = x_ref[pl.ds(e * per, per), :]
    o = b2_ref[pl.ds(e, 1), :]
    for q in range(NSPLIT):
        w1q = w_refs[2 * q]
        w2q = w_refs[2 * q + 1]
        h = jnp.dot(x, w1q[0], preferred_element_type=jnp.float32)
        h = h + b1_ref[pl.ds(e, 1), q * CHUNK:(q + 1) * CHUNK]
        o = o + jnp.dot(h, w2q[0], preferred_element_type=jnp.float32)
    o_ref[pl.ds(e * per, per), :] = o


def kernel(hidden_states, W1, b1, W2, b2):
    Bb, Ss, Hh = hidden_states.shape
    Ee = W1.shape[0]
    T = Bb * Ss
    x = hidden_states.reshape(T, Hh)

    w_specs = []
    w_args = []
    for q in range(NSPLIT):
        w_specs.append(
            pl.BlockSpec((1, Hh, CHUNK), lambda e, q=q: (e, 0, q)))
        w_args.append(W1)
        w_specs.append(
            pl.BlockSpec((1, CHUNK, Hh), lambda e, q=q: (e, q, 0)))
        w_args.append(W2)

    out = pl.pallas_call(
        _ffn_block_kernel,
        grid=(Ee,),
        in_specs=[
            pl.BlockSpec((T, Hh), lambda e: (0, 0)),
            pl.BlockSpec((Ee, I), lambda e: (0, 0)),
            pl.BlockSpec((Ee, Hh), lambda e: (0, 0)),
        ] + w_specs,
        out_specs=pl.BlockSpec((T, Hh), lambda e: (0, 0)),
        out_shape=jax.ShapeDtypeStruct((T, Hh), jnp.float32),
        compiler_params=pltpu.CompilerParams(
            dimension_semantics=("arbitrary",),
        ),
    )(x, b1, b2, *w_args)
    return out.reshape(Bb, Ss, Hh)
